# baseline (speedup 1.0000x reference)
import jax
import jax.numpy as jnp
from jax.experimental import pallas as pl
from jax.experimental.pallas import tpu as pltpu

_NUM_CLASSES = 19
_IGNORE_INDEX = 255
_LANES = 128
_NBLK = 4096


def _pool_kernel(gt_ref, f_ref, o_ref, acc_ref, cnt_ref):
    j = pl.program_id(1)
    nj = pl.num_programs(1)

    @pl.when(j == 0)
    def _():
        acc_ref[...] = jnp.zeros_like(acc_ref)
        cnt_ref[...] = jnp.zeros_like(cnt_ref)

    gt = gt_ref[0]
    valid = gt != _IGNORE_INDEX
    cls = jnp.clip(gt, 0, _NUM_CLASSES - 1)
    lane = jax.lax.broadcasted_iota(jnp.int32, (_NBLK, _LANES), 1)
    onehot = ((cls == lane) & valid).astype(jnp.float32)

    f = f_ref[0]
    acc_ref[...] += jnp.dot(f, onehot, preferred_element_type=jnp.float32)
    cnt_ref[...] += jnp.sum(onehot, axis=0, keepdims=True)

    @pl.when(j == nj - 1)
    def _():
        cnt = cnt_ref[...]
        norm = jnp.where(cnt > 0.0, cnt, 1.0)
        o_ref[0] = acc_ref[...] / norm


def kernel(feats, gt_seg_map):
    B, C, H, W = feats.shape
    HW = H * W
    f = feats.reshape(B, C, HW)
    gt = gt_seg_map.astype(jnp.int32).reshape(B, HW, 1)

    out = pl.pallas_call(
        _pool_kernel,
        grid=(B, HW // _NBLK),
        in_specs=[
            pl.BlockSpec((1, _NBLK, 1), lambda b, j: (b, j, 0)),
            pl.BlockSpec((1, C, _NBLK), lambda b, j: (b, 0, j)),
        ],
        out_specs=pl.BlockSpec((1, C, _LANES), lambda b, j: (b, 0, 0)),
        out_shape=jax.ShapeDtypeStruct((B, C, _LANES), jnp.float32),
        scratch_shapes=[
            pltpu.VMEM((C, _LANES), jnp.float32),
            pltpu.VMEM((1, _LANES), jnp.float32),
        ],
        compiler_params=pltpu.CompilerParams(
            dimension_semantics=("parallel", "arbitrary"),
        ),
        name="class_mean_pool",
    )(gt, f)

    return out[:, :, :_NUM_CLASSES, None]
